# baseline (speedup 1.0000x reference)
import jax
import jax.numpy as jnp
from jax import lax
from jax.experimental import pallas as pl
from jax.experimental.pallas import tpu as pltpu
from jax.experimental.pallas import tpu_sc as plsc

NC = 2
NS = 16
LANES = 16

CHUNK = 128
D = 64


def _sc_kernel_body(widx_hbm, pidx_hbm, word_hbm, pos_hbm, out_hbm,
                    wi_v, pi_v, wu_v, pu_v, wh_v, ph_v,
                    wbuf0, wbuf1, pbuf0, pbuf1, obuf0, obuf1,
                    semw0, semw1, semp0, semp1, sems0, sems1):
    wid = lax.axis_index("c") * NS + lax.axis_index("s")
    n_per_tile = widx_hbm.shape[0] // (NC * NS)
    n_chunks = n_per_tile // CHUNK
    base = wid * n_per_tile

    pltpu.sync_copy(widx_hbm.at[pl.ds(base, n_per_tile)], wi_v)
    pltpu.sync_copy(pidx_hbm.at[pl.ds(base, n_per_tile)], pi_v)

    def idx_body(i, _):
        sl = pl.ds(i * LANES, LANES)
        w = wi_v[sl]
        p = pi_v[sl]
        wu_v[sl] = lax.shift_right_logical(w, 1)
        pu_v[sl] = lax.shift_right_logical(p, 1)
        wh_v[sl] = lax.shift_left(lax.bitwise_and(w, 1), 6)
        ph_v[sl] = lax.shift_left(lax.bitwise_and(p, 1), 6)
        return 0

    lax.fori_loop(0, n_per_tile // LANES, idx_body, 0, unroll=False)

    wbufs = [wbuf0, wbuf1]
    pbufs = [pbuf0, pbuf1]
    obufs = [obuf0, obuf1]
    semws = [semw0, semw1]
    semps = [semp0, semp1]
    semss = [sems0, sems1]
    cw = [None, None]
    cp = [None, None]
    cs = [None, None]

    def issue_chunk(k, b):
        sl = pl.ds(k * CHUNK, CHUNK)
        cw[b] = pltpu.async_copy(word_hbm.at[wu_v.at[sl]], wbufs[b], semws[b])
        cp[b] = pltpu.async_copy(pos_hbm.at[pu_v.at[sl]], pbufs[b], semps[b])

    issue_chunk(0, 0)
    for k in range(n_chunks):
        b = k % 2
        nb = (k + 1) % 2
        cw[b].wait()
        cp[b].wait()
        if k + 1 < n_chunks:
            if cs[nb] is not None:
                cs[nb].wait()
            issue_chunk(k + 1, nb)

        wbuf, pbuf, obuf = wbufs[b], pbufs[b], obufs[b]
        koff = k * CHUNK

        def extract_body(r, _):
            hw = wh_v[pl.ds(koff + r, LANES)][0]
            hp = ph_v[pl.ds(koff + r, LANES)][0]
            for c in range(D // LANES):
                co = c * LANES
                obuf[r, pl.ds(co, LANES)] = (
                    wbuf[r, pl.ds(hw + co, LANES)]
                    + pbuf[r, pl.ds(hp + co, LANES)])
            return 0

        lax.fori_loop(0, CHUNK, extract_body, 0, unroll=False)
        cs[b] = pltpu.async_copy(
            obuf, out_hbm.at[pl.ds(base + k * CHUNK, CHUNK)], semss[b])

    for c in cs:
        if c is not None:
            c.wait()


def kernel(src_word, src_pos, word_emb, pos_emb):
    B, S = src_word.shape
    n = B * S
    V, _ = word_emb.shape
    P, _ = pos_emb.shape
    widx = src_word.reshape(n)
    pidx = src_pos.reshape(n)
    w2 = word_emb.reshape(V // 2, 2 * D)
    p2 = pos_emb.reshape(P // 2, 2 * D)

    mesh = plsc.VectorSubcoreMesh(core_axis_name="c", subcore_axis_name="s",
                                  num_cores=NC, num_subcores=NS)
    n_per_tile = n // (NC * NS)
    run = pl.kernel(
        _sc_kernel_body,
        out_type=jax.ShapeDtypeStruct((n, D), jnp.float32),
        mesh=mesh,
        compiler_params=pltpu.CompilerParams(use_tc_tiling_on_sc=True),
        scratch_types=[
            pltpu.VMEM((n_per_tile,), jnp.int32),
            pltpu.VMEM((n_per_tile,), jnp.int32),
            pltpu.VMEM((n_per_tile,), jnp.int32),
            pltpu.VMEM((n_per_tile,), jnp.int32),
            pltpu.VMEM((n_per_tile + LANES,), jnp.int32),
            pltpu.VMEM((n_per_tile + LANES,), jnp.int32),
            pltpu.VMEM((CHUNK, 2 * D), jnp.float32),
            pltpu.VMEM((CHUNK, 2 * D), jnp.float32),
            pltpu.VMEM((CHUNK, 2 * D), jnp.float32),
            pltpu.VMEM((CHUNK, 2 * D), jnp.float32),
            pltpu.VMEM((CHUNK, D), jnp.float32),
            pltpu.VMEM((CHUNK, D), jnp.float32),
            pltpu.SemaphoreType.DMA,
            pltpu.SemaphoreType.DMA,
            pltpu.SemaphoreType.DMA,
            pltpu.SemaphoreType.DMA,
            pltpu.SemaphoreType.DMA,
            pltpu.SemaphoreType.DMA,
        ],
    )
    out = run(widx, pidx, w2, p2)
    return out.reshape(B, S, D)


# speedup vs baseline: 1.6354x; 1.6354x over previous
import jax
import jax.numpy as jnp
from jax import lax
from jax.experimental import pallas as pl
from jax.experimental.pallas import tpu as pltpu
from jax.experimental.pallas import tpu_sc as plsc

NC = 2
NS = 16
LANES = 16

CHUNK = 128
D = 64


def _sc_kernel_body(widx_hbm, pidx_hbm, word_hbm, pos_hbm, out_hbm,
                    wi_v, pi_v, wbuf0, wbuf1, pbuf0, pbuf1,
                    semw0, semw1, semp0, semp1, sems0, sems1):
    wid = lax.axis_index("c") * NS + lax.axis_index("s")
    n_per_tile = widx_hbm.shape[0] // (NC * NS)
    n_chunks = n_per_tile // CHUNK
    base = wid * n_per_tile

    pltpu.sync_copy(widx_hbm.at[pl.ds(base, n_per_tile)],
                    wi_v.at[pl.ds(0, n_per_tile)])
    pltpu.sync_copy(pidx_hbm.at[pl.ds(base, n_per_tile)],
                    pi_v.at[pl.ds(0, n_per_tile)])

    wbufs = [wbuf0, wbuf1]
    pbufs = [pbuf0, pbuf1]
    semws = [semw0, semw1]
    semps = [semp0, semp1]
    semss = [sems0, sems1]
    cs = [None, None]

    def issue_chunk(k, b):
        koff = k * CHUNK
        wbuf, pbuf = wbufs[b], pbufs[b]
        semw, semp = semws[b], semps[b]

        def issue_body(r, _):
            iw = wi_v[pl.ds(koff + r, LANES)][0]
            ip = pi_v[pl.ds(koff + r, LANES)][0]
            pltpu.async_copy(word_hbm.at[iw], wbuf.at[r], semw)
            pltpu.async_copy(pos_hbm.at[ip], pbuf.at[r], semp)
            return 0

        lax.fori_loop(0, CHUNK, issue_body, 0, unroll=False)

    def drain_chunk(b):
        wbuf, pbuf = wbufs[b], pbufs[b]
        semw, semp = semws[b], semps[b]

        def drain_body(r, _):
            pltpu.make_async_copy(word_hbm.at[0], wbuf.at[r], semw).wait()
            pltpu.make_async_copy(pos_hbm.at[0], pbuf.at[r], semp).wait()
            return 0

        lax.fori_loop(0, CHUNK, drain_body, 0, unroll=False)

    issue_chunk(0, 0)

    for k in range(n_chunks):
        b = k % 2
        nb = (k + 1) % 2
        if k + 1 < n_chunks:
            if cs[nb] is not None:
                cs[nb].wait()
            issue_chunk(k + 1, nb)
        drain_chunk(b)

        wbuf, pbuf = wbufs[b], pbufs[b]

        def add_body(r, _):
            for c in range(D // LANES):
                sl = pl.ds(c * LANES, LANES)
                wbuf[r, sl] = wbuf[r, sl] + pbuf[r, sl]
            return 0

        lax.fori_loop(0, CHUNK, add_body, 0, unroll=False)
        cs[b] = pltpu.async_copy(
            wbuf, out_hbm.at[pl.ds(base + k * CHUNK, CHUNK)], semss[b])

    for c in cs:
        if c is not None:
            c.wait()


def kernel(src_word, src_pos, word_emb, pos_emb):
    B, S = src_word.shape
    n = B * S
    widx = src_word.reshape(n)
    pidx = src_pos.reshape(n)

    mesh = plsc.VectorSubcoreMesh(core_axis_name="c", subcore_axis_name="s",
                                  num_cores=NC, num_subcores=NS)
    n_per_tile = n // (NC * NS)
    run = pl.kernel(
        _sc_kernel_body,
        out_type=jax.ShapeDtypeStruct((n, D), jnp.float32),
        mesh=mesh,
        compiler_params=pltpu.CompilerParams(use_tc_tiling_on_sc=True),
        scratch_types=[
            pltpu.VMEM((n_per_tile + LANES,), jnp.int32),
            pltpu.VMEM((n_per_tile + LANES,), jnp.int32),
            pltpu.VMEM((CHUNK, D), jnp.float32),
            pltpu.VMEM((CHUNK, D), jnp.float32),
            pltpu.VMEM((CHUNK, D), jnp.float32),
            pltpu.VMEM((CHUNK, D), jnp.float32),
            pltpu.SemaphoreType.DMA,
            pltpu.SemaphoreType.DMA,
            pltpu.SemaphoreType.DMA,
            pltpu.SemaphoreType.DMA,
            pltpu.SemaphoreType.DMA,
            pltpu.SemaphoreType.DMA,
        ],
    )
    out = run(widx, pidx, word_emb, pos_emb)
    return out.reshape(B, S, D)
